# Initial kernel scaffold; baseline (speedup 1.0000x reference)
import jax
import jax.numpy as jnp
from jax.experimental import pallas as pl


def kernel(src, flow):
    raise NotImplementedError("write your pallas kernel here")


# speedup vs baseline: 1.4807x; 1.4807x over previous
import functools

import jax
import jax.numpy as jnp
from jax import lax
from jax.experimental import pallas as pl
from jax.experimental.pallas import tpu as pltpu
from jax.experimental.pallas import tpu_sc as plsc

H = 384
W = 384
B = 2
C = 96
N = B * H * W
NW = 32
PIX_PER_W = N // NW
CHUNK = 128
NCHUNK = PIX_PER_W // CHUNK
LANES = 16

_INV = 2.0 / W


def _warp_body(src_hbm, fx_hbm, fy_hbm, out_hbm,
               fx_v, fy_v,
               i00, i01, i10, i11,
               w00, w01, w10, w11,
               t00, t01, t10, t11,
               out_v, sem):
    wid = lax.axis_index("s") * 2 + lax.axis_index("c")
    worker_base = wid * PIX_PER_W

    def chunk_body(ci, carry):
        base = worker_base + ci * CHUNK
        pltpu.sync_copy(fx_hbm.at[pl.ds(base, CHUNK)], fx_v)
        pltpu.sync_copy(fy_hbm.at[pl.ds(base, CHUNK)], fy_v)

        def idx_body(j, c2):
            o = j * LANES
            sl = pl.ds(o, LANES)
            row = base // W
            col0 = base % W
            bi = row // H
            yi = row % H
            xi = col0 + o + lax.iota(jnp.int32, LANES)
            gx = (xi.astype(jnp.float32) + 0.5) * _INV - 1.0 + fx_v[sl] * _INV
            gy = (jnp.float32(yi) + 0.5) * _INV - 1.0 + fy_v[sl] * _INV
            ix = ((gx + 1.0) * W - 1.0) * 0.5
            iy = ((gy + 1.0) * H - 1.0) * 0.5
            ix = jnp.minimum(jnp.maximum(ix, 0.0), jnp.float32(W - 1))
            iy = jnp.minimum(jnp.maximum(iy, 0.0), jnp.float32(H - 1))
            ix0 = ix.astype(jnp.int32)
            iy0 = iy.astype(jnp.int32)
            wx1 = ix - ix0.astype(jnp.float32)
            wy1 = iy - iy0.astype(jnp.float32)
            wx0 = 1.0 - wx1
            wy0 = 1.0 - wy1
            ix1 = jnp.minimum(ix0 + 1, W - 1)
            iy1 = jnp.minimum(iy0 + 1, H - 1)
            row0 = bi * (H * W) + iy0 * W
            row1 = bi * (H * W) + iy1 * W
            i00[sl] = row0 + ix0
            i01[sl] = row0 + ix1
            i10[sl] = row1 + ix0
            i11[sl] = row1 + ix1
            w00[sl] = wy0 * wx0
            w01[sl] = wy0 * wx1
            w10[sl] = wy1 * wx0
            w11[sl] = wy1 * wx1
            return c2

        lax.fori_loop(0, CHUNK // LANES, idx_body, 0, unroll=False)

        if True:
            c0 = pltpu.async_copy(src_hbm.at[i00], t00, sem)
            c1 = pltpu.async_copy(src_hbm.at[i01], t01, sem)
            c2 = pltpu.async_copy(src_hbm.at[i10], t10, sem)
            c3 = pltpu.async_copy(src_hbm.at[i11], t11, sem)
            c0.wait()
            c1.wait()
            c2.wait()
            c3.wait()

        def px_body(g, c2):
            gsl = pl.ds(g * LANES, LANES)
            w00v = w00[gsl]
            w01v = w01[gsl]
            w10v = w10[gsl]
            w11v = w11[gsl]
            for jj in range(LANES):
                p = g * LANES + jj
                b00 = jnp.full((LANES,), w00v[jj], jnp.float32)
                b01 = jnp.full((LANES,), w01v[jj], jnp.float32)
                b10 = jnp.full((LANES,), w10v[jj], jnp.float32)
                b11 = jnp.full((LANES,), w11v[jj], jnp.float32)
                for cc in range(C // LANES):
                    sl = pl.ds(cc * LANES, LANES)
                    out_v[p, sl] = (t00[p, sl] * b00 + t01[p, sl] * b01
                                    + t10[p, sl] * b10 + t11[p, sl] * b11)
            return c2

        lax.fori_loop(0, CHUNK // LANES, px_body, 0, unroll=False)

        pltpu.sync_copy(out_v, out_hbm.at[pl.ds(base, CHUNK)])
        return carry

    lax.fori_loop(0, NCHUNK, chunk_body, 0, unroll=False)


@jax.jit
def _warp(src_cl, fx, fy):
    mesh = plsc.VectorSubcoreMesh(core_axis_name="c", subcore_axis_name="s",
                                  num_cores=2, num_subcores=16)
    f = functools.partial(
        pl.kernel,
        out_type=jax.ShapeDtypeStruct((N, C), jnp.float32),
        mesh=mesh,
        compiler_params=pltpu.CompilerParams(use_tc_tiling_on_sc=False),
        scratch_types=[
            pltpu.VMEM((CHUNK,), jnp.float32),
            pltpu.VMEM((CHUNK,), jnp.float32),
            pltpu.VMEM((CHUNK,), jnp.int32),
            pltpu.VMEM((CHUNK,), jnp.int32),
            pltpu.VMEM((CHUNK,), jnp.int32),
            pltpu.VMEM((CHUNK,), jnp.int32),
            pltpu.VMEM((CHUNK,), jnp.float32),
            pltpu.VMEM((CHUNK,), jnp.float32),
            pltpu.VMEM((CHUNK,), jnp.float32),
            pltpu.VMEM((CHUNK,), jnp.float32),
            pltpu.VMEM((CHUNK, C), jnp.float32),
            pltpu.VMEM((CHUNK, C), jnp.float32),
            pltpu.VMEM((CHUNK, C), jnp.float32),
            pltpu.VMEM((CHUNK, C), jnp.float32),
            pltpu.VMEM((CHUNK, C), jnp.float32),
            pltpu.SemaphoreType.DMA,
        ],
    )(_warp_body)
    return f(src_cl, fx, fy)


def kernel(src, flow):
    src_cl = src.transpose(0, 2, 3, 1).reshape(N, C)
    fx = flow[..., 0].reshape(N)
    fy = flow[..., 1].reshape(N)
    out_cl = _warp(src_cl, fx, fy)
    return out_cl.reshape(B, H, W, C).transpose(0, 3, 1, 2)


# speedup vs baseline: 1.7442x; 1.1779x over previous
import functools

import jax
import jax.numpy as jnp
from jax import lax
from jax.experimental import pallas as pl
from jax.experimental.pallas import tpu as pltpu
from jax.experimental.pallas import tpu_sc as plsc

H = 384
W = 384
B = 2
C = 96
N = B * H * W
NW = 32
PIX_PER_W = N // NW
CHUNK = 96
NCHUNK = PIX_PER_W // CHUNK
LANES = 16

_INV = 2.0 / W


def _warp_body(src_hbm, fx_hbm, fy_hbm, out_hbm,
               flow_v, idx_v, wgt_v, taps_v, out_v,
               gsem0, gsem1, osem0, osem1):
    gsems = (gsem0, gsem1)
    osems = (osem0, osem1)
    wid = lax.axis_index("s") * 2 + lax.axis_index("c")
    worker_base = wid * PIX_PER_W

    def stage(ci, b):
        base = worker_base + ci * CHUNK
        pltpu.sync_copy(fx_hbm.at[pl.ds(base, CHUNK)], flow_v.at[b, 0])
        pltpu.sync_copy(fy_hbm.at[pl.ds(base, CHUNK)], flow_v.at[b, 1])

        def idx_body(j, c2):
            o = j * LANES
            sl = pl.ds(o, LANES)
            row = base // W
            col0 = base % W
            bi = row // H
            yi = row % H
            xi = col0 + o + lax.iota(jnp.int32, LANES)
            gx = (xi.astype(jnp.float32) + 0.5) * _INV - 1.0 + flow_v[b, 0, sl] * _INV
            gy = (jnp.float32(yi) + 0.5) * _INV - 1.0 + flow_v[b, 1, sl] * _INV
            ix = ((gx + 1.0) * W - 1.0) * 0.5
            iy = ((gy + 1.0) * H - 1.0) * 0.5
            ix = jnp.minimum(jnp.maximum(ix, 0.0), jnp.float32(W - 1))
            iy = jnp.minimum(jnp.maximum(iy, 0.0), jnp.float32(H - 1))
            ix0 = ix.astype(jnp.int32)
            iy0 = iy.astype(jnp.int32)
            wx1 = ix - ix0.astype(jnp.float32)
            wy1 = iy - iy0.astype(jnp.float32)
            wx0 = 1.0 - wx1
            wy0 = 1.0 - wy1
            ix1 = jnp.minimum(ix0 + 1, W - 1)
            iy1 = jnp.minimum(iy0 + 1, H - 1)
            row0 = bi * (H * W) + iy0 * W
            row1 = bi * (H * W) + iy1 * W
            idx_v[b, 0, sl] = row0 + ix0
            idx_v[b, 1, sl] = row0 + ix1
            idx_v[b, 2, sl] = row1 + ix0
            idx_v[b, 3, sl] = row1 + ix1
            wgt_v[b, 0, sl] = wy0 * wx0
            wgt_v[b, 1, sl] = wy0 * wx1
            wgt_v[b, 2, sl] = wy1 * wx0
            wgt_v[b, 3, sl] = wy1 * wx1
            return c2

        lax.fori_loop(0, CHUNK // LANES, idx_body, 0, unroll=False)
        for k in range(4):
            pltpu.async_copy(src_hbm.at[idx_v.at[b, k]], taps_v.at[b, k],
                             gsems[b])

    def drain_gathers(b):
        for k in range(4):
            pltpu.make_async_copy(src_hbm.at[idx_v.at[b, k]], taps_v.at[b, k],
                                  gsems[b]).wait()

    def combine(ci, b):
        base = worker_base + ci * CHUNK

        def px_body(g, c2):
            gsl = pl.ds(g * LANES, LANES)
            w00v = wgt_v[b, 0, gsl]
            w01v = wgt_v[b, 1, gsl]
            w10v = wgt_v[b, 2, gsl]
            w11v = wgt_v[b, 3, gsl]
            for jj in range(LANES):
                p = g * LANES + jj
                b00 = jnp.full((LANES,), w00v[jj], jnp.float32)
                b01 = jnp.full((LANES,), w01v[jj], jnp.float32)
                b10 = jnp.full((LANES,), w10v[jj], jnp.float32)
                b11 = jnp.full((LANES,), w11v[jj], jnp.float32)
                for cc in range(C // LANES):
                    sl = pl.ds(cc * LANES, LANES)
                    out_v[b, p, sl] = (
                        taps_v[b, 0, p, sl] * b00 + taps_v[b, 1, p, sl] * b01
                        + taps_v[b, 2, p, sl] * b10 + taps_v[b, 3, p, sl] * b11)
            return c2

        lax.fori_loop(0, CHUNK // LANES, px_body, 0, unroll=False)
        pltpu.async_copy(out_v.at[b], out_hbm.at[pl.ds(base, CHUNK)],
                         osems[b])

    def drain_store(b):
        pltpu.make_async_copy(out_v.at[b], out_hbm.at[pl.ds(0, CHUNK)],
                              osems[b]).wait()

    stage(0, 0)

    def pair_body(i, carry):
        for b in range(2):
            ci = 2 * i + b
            nb = 1 - b

            @pl.when(ci + 1 < NCHUNK)
            def _():
                stage(ci + 1, nb)

            drain_gathers(b)

            @pl.when(ci >= 2)
            def _():
                drain_store(b)

            combine(ci, b)
        return carry

    lax.fori_loop(0, NCHUNK // 2, pair_body, 0, unroll=False)
    drain_store(0)
    drain_store(1)


@jax.jit
def _warp(src_cl, fx, fy):
    mesh = plsc.VectorSubcoreMesh(core_axis_name="c", subcore_axis_name="s",
                                  num_cores=2, num_subcores=16)
    f = functools.partial(
        pl.kernel,
        out_type=jax.ShapeDtypeStruct((N, C), jnp.float32),
        mesh=mesh,
        compiler_params=pltpu.CompilerParams(use_tc_tiling_on_sc=False),
        scratch_types=[
            pltpu.VMEM((2, 2, CHUNK), jnp.float32),
            pltpu.VMEM((2, 4, CHUNK), jnp.int32),
            pltpu.VMEM((2, 4, CHUNK), jnp.float32),
            pltpu.VMEM((2, 4, CHUNK, C), jnp.float32),
            pltpu.VMEM((2, CHUNK, C), jnp.float32),
            pltpu.SemaphoreType.DMA,
            pltpu.SemaphoreType.DMA,
            pltpu.SemaphoreType.DMA,
            pltpu.SemaphoreType.DMA,
        ],
    )(_warp_body)
    return f(src_cl, fx, fy)


def kernel(src, flow):
    src_cl = src.transpose(0, 2, 3, 1).reshape(N, C)
    fx = flow[..., 0].reshape(N)
    fy = flow[..., 1].reshape(N)
    out_cl = _warp(src_cl, fx, fy)
    return out_cl.reshape(B, H, W, C).transpose(0, 3, 1, 2)


# speedup vs baseline: 1.9461x; 1.1158x over previous
import functools

import jax
import jax.numpy as jnp
from jax import lax
from jax.experimental import pallas as pl
from jax.experimental.pallas import tpu as pltpu
from jax.experimental.pallas import tpu_sc as plsc

H = 384
W = 384
B = 2
C = 96
CP = 128
N = B * H * W
NW = 32
PIX_PER_W = N // NW
CHUNK = 64
NCHUNK = PIX_PER_W // CHUNK
LANES = 16

_INV = 2.0 / W


def _warp_body(src_hbm, fx_hbm, fy_hbm, out_hbm,
               flow_v, idx_v, wgt_v, taps_v, out_v,
               gsem0, gsem1, osem0, osem1):
    gsems = (gsem0, gsem1)
    osems = (osem0, osem1)
    wid = lax.axis_index("s") * 2 + lax.axis_index("c")
    worker_base = wid * PIX_PER_W

    def stage(ci, b):
        base = worker_base + ci * CHUNK
        pltpu.sync_copy(fx_hbm.at[pl.ds(base, CHUNK)], flow_v.at[b, 0])
        pltpu.sync_copy(fy_hbm.at[pl.ds(base, CHUNK)], flow_v.at[b, 1])

        def idx_body(j, c2):
            o = j * LANES
            sl = pl.ds(o, LANES)
            row = base // W
            col0 = base % W
            bi = row // H
            yi = row % H
            xi = col0 + o + lax.iota(jnp.int32, LANES)
            gx = (xi.astype(jnp.float32) + 0.5) * _INV - 1.0 + flow_v[b, 0, sl] * _INV
            gy = (jnp.float32(yi) + 0.5) * _INV - 1.0 + flow_v[b, 1, sl] * _INV
            ix = ((gx + 1.0) * W - 1.0) * 0.5
            iy = ((gy + 1.0) * H - 1.0) * 0.5
            ix = jnp.minimum(jnp.maximum(ix, 0.0), jnp.float32(W - 1))
            iy = jnp.minimum(jnp.maximum(iy, 0.0), jnp.float32(H - 1))
            ix0 = ix.astype(jnp.int32)
            iy0 = iy.astype(jnp.int32)
            wx1 = ix - ix0.astype(jnp.float32)
            wy1 = iy - iy0.astype(jnp.float32)
            wx0 = 1.0 - wx1
            wy0 = 1.0 - wy1
            ix1 = jnp.minimum(ix0 + 1, W - 1)
            iy1 = jnp.minimum(iy0 + 1, H - 1)
            row0 = bi * (H * W) + iy0 * W
            row1 = bi * (H * W) + iy1 * W
            idx_v[b, 0, sl] = row0 + ix0
            idx_v[b, 1, sl] = row0 + ix1
            idx_v[b, 2, sl] = row1 + ix0
            idx_v[b, 3, sl] = row1 + ix1
            wgt_v[b, 0, sl] = wy0 * wx0
            wgt_v[b, 1, sl] = wy0 * wx1
            wgt_v[b, 2, sl] = wy1 * wx0
            wgt_v[b, 3, sl] = wy1 * wx1
            return c2

        lax.fori_loop(0, CHUNK // LANES, idx_body, 0, unroll=False)
        for k in range(4):
            pltpu.async_copy(src_hbm.at[idx_v.at[b, k]], taps_v.at[b, k],
                             gsems[b])

    def drain_gathers(b):
        for k in range(4):
            pltpu.make_async_copy(src_hbm.at[idx_v.at[b, k]], taps_v.at[b, k],
                                  gsems[b]).wait()

    def combine(ci, b):
        base = worker_base + ci * CHUNK

        def px_body(g, c2):
            gsl = pl.ds(g * LANES, LANES)
            w00v = wgt_v[b, 0, gsl]
            w01v = wgt_v[b, 1, gsl]
            w10v = wgt_v[b, 2, gsl]
            w11v = wgt_v[b, 3, gsl]
            for jj in range(LANES):
                p = g * LANES + jj
                b00 = jnp.full((LANES,), w00v[jj], jnp.float32)
                b01 = jnp.full((LANES,), w01v[jj], jnp.float32)
                b10 = jnp.full((LANES,), w10v[jj], jnp.float32)
                b11 = jnp.full((LANES,), w11v[jj], jnp.float32)
                for cc in range(C // LANES):
                    sl = pl.ds(cc * LANES, LANES)
                    out_v[b, p, sl] = (
                        taps_v[b, 0, p, sl] * b00 + taps_v[b, 1, p, sl] * b01
                        + taps_v[b, 2, p, sl] * b10 + taps_v[b, 3, p, sl] * b11)
            return c2

        lax.fori_loop(0, CHUNK // LANES, px_body, 0, unroll=False)
        pltpu.async_copy(out_v.at[b], out_hbm.at[pl.ds(base, CHUNK)],
                         osems[b])

    def drain_store(b):
        pltpu.make_async_copy(out_v.at[b], out_hbm.at[pl.ds(0, CHUNK)],
                              osems[b]).wait()

    stage(0, 0)

    def pair_body(i, carry):
        for b in range(2):
            ci = 2 * i + b
            nb = 1 - b

            @pl.when(ci + 1 < NCHUNK)
            def _():
                stage(ci + 1, nb)

            drain_gathers(b)

            @pl.when(ci >= 2)
            def _():
                drain_store(b)

            combine(ci, b)
        return carry

    lax.fori_loop(0, NCHUNK // 2, pair_body, 0, unroll=False)
    drain_store(0)
    drain_store(1)


@jax.jit
def _warp(src_cl, fx, fy):
    mesh = plsc.VectorSubcoreMesh(core_axis_name="c", subcore_axis_name="s",
                                  num_cores=2, num_subcores=16)
    f = functools.partial(
        pl.kernel,
        out_type=jax.ShapeDtypeStruct((N, CP), jnp.float32),
        mesh=mesh,
        scratch_types=[
            pltpu.VMEM((2, 2, CHUNK), jnp.float32),
            pltpu.VMEM((2, 4, CHUNK), jnp.int32),
            pltpu.VMEM((2, 4, CHUNK), jnp.float32),
            pltpu.VMEM((2, 4, CHUNK, CP), jnp.float32),
            pltpu.VMEM((2, CHUNK, CP), jnp.float32),
            pltpu.SemaphoreType.DMA,
            pltpu.SemaphoreType.DMA,
            pltpu.SemaphoreType.DMA,
            pltpu.SemaphoreType.DMA,
        ],
    )(_warp_body)
    return f(src_cl, fx, fy)


def kernel(src, flow):
    src_p = jnp.pad(src, ((0, 0), (0, CP - C), (0, 0), (0, 0)))
    src_cl = src_p.transpose(0, 2, 3, 1).reshape(N, CP)
    fx = flow[..., 0].reshape(N)
    fy = flow[..., 1].reshape(N)
    out_cl = _warp(src_cl, fx, fy)
    return out_cl.reshape(B, H, W, CP).transpose(0, 3, 1, 2)[:, :C]


# speedup vs baseline: 2.1728x; 1.1165x over previous
import functools

import jax
import jax.numpy as jnp
from jax import lax
from jax.experimental import pallas as pl
from jax.experimental.pallas import tpu as pltpu
from jax.experimental.pallas import tpu_sc as plsc

H = 384
W = 384
B = 2
C = 96
CP = 128
N = B * H * W
NW = 32
PIX_PER_W = N // NW
CHUNK = 64
NCHUNK = PIX_PER_W // CHUNK
LANES = 16

_INV = 2.0 / W


def _warp_body(src_hbm, fx_hbm, fy_hbm, out_hbm,
               flow_v, idx_v, wgt_v, taps_v, out_v,
               gsem0, gsem1, osem0, osem1):
    gsems = (gsem0, gsem1)
    osems = (osem0, osem1)
    wid = lax.axis_index("s") * 2 + lax.axis_index("c")
    worker_base = wid * PIX_PER_W

    def stage(ci, b):
        base = worker_base + ci * CHUNK
        pltpu.sync_copy(fx_hbm.at[pl.ds(base, CHUNK)], flow_v.at[b, 0])
        pltpu.sync_copy(fy_hbm.at[pl.ds(base, CHUNK)], flow_v.at[b, 1])

        def idx_body(j, c2):
            o = j * LANES
            sl = pl.ds(o, LANES)
            row = base // W
            col0 = base % W
            bi = row // H
            yi = row % H
            xi = col0 + o + lax.iota(jnp.int32, LANES)
            gx = (xi.astype(jnp.float32) + 0.5) * _INV - 1.0 + flow_v[b, 0, sl] * _INV
            gy = (jnp.float32(yi) + 0.5) * _INV - 1.0 + flow_v[b, 1, sl] * _INV
            ix = ((gx + 1.0) * W - 1.0) * 0.5
            iy = ((gy + 1.0) * H - 1.0) * 0.5
            ix = jnp.minimum(jnp.maximum(ix, 0.0), jnp.float32(W - 1))
            iy = jnp.minimum(jnp.maximum(iy, 0.0), jnp.float32(H - 1))
            ix0 = ix.astype(jnp.int32)
            iy0 = iy.astype(jnp.int32)
            wx1 = ix - ix0.astype(jnp.float32)
            wy1 = iy - iy0.astype(jnp.float32)
            wx0 = 1.0 - wx1
            wy0 = 1.0 - wy1
            ix1 = jnp.minimum(ix0 + 1, W - 1)
            iy1 = jnp.minimum(iy0 + 1, H - 1)
            row0 = bi * (H * W) + iy0 * W
            row1 = bi * (H * W) + iy1 * W
            idx_v[b, 0, sl] = row0 + ix0
            idx_v[b, 1, sl] = row0 + ix1
            idx_v[b, 2, sl] = row1 + ix0
            idx_v[b, 3, sl] = row1 + ix1
            wgt_v[b, 0, sl] = wy0 * wx0
            wgt_v[b, 1, sl] = wy0 * wx1
            wgt_v[b, 2, sl] = wy1 * wx0
            wgt_v[b, 3, sl] = wy1 * wx1
            return c2

        lax.fori_loop(0, CHUNK // LANES, idx_body, 0, unroll=False)
        for k in range(4):
            pltpu.async_copy(src_hbm.at[idx_v.at[b, k]], taps_v.at[b, k],
                             gsems[b])

    def drain_gathers(b):
        for k in range(4):
            pltpu.make_async_copy(src_hbm.at[idx_v.at[b, k]], taps_v.at[b, k],
                                  gsems[b]).wait()

    def combine(ci, b):
        base = worker_base + ci * CHUNK

        def px_body(g, c2):
            gsl = pl.ds(g * LANES, LANES)
            w00v = wgt_v[b, 0, gsl]
            w01v = wgt_v[b, 1, gsl]
            w10v = wgt_v[b, 2, gsl]
            w11v = wgt_v[b, 3, gsl]
            for jj in range(LANES):
                p = g * LANES + jj
                b00 = jnp.full((LANES,), w00v[jj], jnp.float32)
                b01 = jnp.full((LANES,), w01v[jj], jnp.float32)
                b10 = jnp.full((LANES,), w10v[jj], jnp.float32)
                b11 = jnp.full((LANES,), w11v[jj], jnp.float32)
                for cc in range(C // LANES):
                    sl = pl.ds(cc * LANES, LANES)
                    out_v[b, p, sl] = (
                        taps_v[b, 0, p, sl] * b00 + taps_v[b, 1, p, sl] * b01
                        + taps_v[b, 2, p, sl] * b10 + taps_v[b, 3, p, sl] * b11)
            return c2

        lax.fori_loop(0, CHUNK // LANES, px_body, 0, unroll=False)
        pltpu.async_copy(out_v.at[b], out_hbm.at[pl.ds(base, CHUNK)],
                         osems[b])

    def drain_store(b):
        pltpu.make_async_copy(out_v.at[b], out_hbm.at[pl.ds(0, CHUNK)],
                              osems[b]).wait()

    stage(0, 0)

    def pair_body(i, carry):
        for b in range(2):
            ci = 2 * i + b
            nb = 1 - b

            @pl.when(ci + 1 < NCHUNK)
            def _():
                stage(ci + 1, nb)

            drain_gathers(b)

            @pl.when(ci >= 2)
            def _():
                drain_store(b)

            combine(ci, b)
        return carry

    lax.fori_loop(0, NCHUNK // 2, pair_body, 0, unroll=False)
    drain_store(0)
    drain_store(1)


@jax.jit
def _warp(src_cl, fx, fy):
    mesh = plsc.VectorSubcoreMesh(core_axis_name="c", subcore_axis_name="s",
                                  num_cores=2, num_subcores=16)
    f = functools.partial(
        pl.kernel,
        out_type=jax.ShapeDtypeStruct((N, C), jnp.float32),
        mesh=mesh,
        scratch_types=[
            pltpu.VMEM((2, 2, CHUNK), jnp.float32),
            pltpu.VMEM((2, 4, CHUNK), jnp.int32),
            pltpu.VMEM((2, 4, CHUNK), jnp.float32),
            pltpu.VMEM((2, 4, CHUNK, CP), jnp.float32),
            pltpu.VMEM((2, CHUNK, C), jnp.float32),
            pltpu.SemaphoreType.DMA,
            pltpu.SemaphoreType.DMA,
            pltpu.SemaphoreType.DMA,
            pltpu.SemaphoreType.DMA,
        ],
    )(_warp_body)
    return f(src_cl, fx, fy)


def kernel(src, flow):
    src_p = jnp.pad(src, ((0, 0), (0, CP - C), (0, 0), (0, 0)))
    src_cl = src_p.transpose(0, 2, 3, 1).reshape(N, CP)
    fx = flow[..., 0].reshape(N)
    fy = flow[..., 1].reshape(N)
    out_cl = _warp(src_cl, fx, fy)
    return out_cl.reshape(B, H, W, C).transpose(0, 3, 1, 2)


# speedup vs baseline: 2.4928x; 1.1473x over previous
import functools

import jax
import jax.numpy as jnp
from jax import lax
from jax.experimental import pallas as pl
from jax.experimental.pallas import tpu as pltpu
from jax.experimental.pallas import tpu_sc as plsc

H = 384
W = 384
B = 2
C = 96
CP = 128
N = B * H * W
NW = 32
PIX_PER_W = N // NW
CHUNK = 64
NCHUNK = PIX_PER_W // CHUNK
LANES = 16

_INV = 2.0 / W


def _warp_body(src_hbm, fx_hbm, fy_hbm, out_hbm,
               fx_sp, fy_sp, idx_v, wgt_v, taps_v, out_v,
               gsem0, gsem1, osem0, osem1):
    gsems = (gsem0, gsem1)
    osems = (osem0, osem1)
    wid = lax.axis_index("s") * 2 + lax.axis_index("c")
    worker_base = wid * PIX_PER_W
    pltpu.sync_copy(fx_hbm.at[pl.ds(worker_base, PIX_PER_W)], fx_sp)
    pltpu.sync_copy(fy_hbm.at[pl.ds(worker_base, PIX_PER_W)], fy_sp)

    def stage(ci, b):
        base = worker_base + ci * CHUNK

        def idx_body(j, c2):
            o = j * LANES
            sl = pl.ds(o, LANES)
            fsl = pl.ds(ci * CHUNK + o, LANES)
            row = base // W
            col0 = base % W
            bi = row // H
            yi = row % H
            xi = col0 + o + lax.iota(jnp.int32, LANES)
            gx = (xi.astype(jnp.float32) + 0.5) * _INV - 1.0 + fx_sp[fsl] * _INV
            gy = (jnp.float32(yi) + 0.5) * _INV - 1.0 + fy_sp[fsl] * _INV
            ix = ((gx + 1.0) * W - 1.0) * 0.5
            iy = ((gy + 1.0) * H - 1.0) * 0.5
            ix = jnp.minimum(jnp.maximum(ix, 0.0), jnp.float32(W - 1))
            iy = jnp.minimum(jnp.maximum(iy, 0.0), jnp.float32(H - 1))
            ix0 = ix.astype(jnp.int32)
            iy0 = iy.astype(jnp.int32)
            wx1 = ix - ix0.astype(jnp.float32)
            wy1 = iy - iy0.astype(jnp.float32)
            wx0 = 1.0 - wx1
            wy0 = 1.0 - wy1
            ix1 = jnp.minimum(ix0 + 1, W - 1)
            iy1 = jnp.minimum(iy0 + 1, H - 1)
            row0 = bi * (H * W) + iy0 * W
            row1 = bi * (H * W) + iy1 * W
            idx_v[b, 0, sl] = row0 + ix0
            idx_v[b, 1, sl] = row0 + ix1
            idx_v[b, 2, sl] = row1 + ix0
            idx_v[b, 3, sl] = row1 + ix1
            wgt_v[b, 0, sl] = wy0 * wx0
            wgt_v[b, 1, sl] = wy0 * wx1
            wgt_v[b, 2, sl] = wy1 * wx0
            wgt_v[b, 3, sl] = wy1 * wx1
            return c2

        lax.fori_loop(0, CHUNK // LANES, idx_body, 0, unroll=False)
        for k in range(4):
            pltpu.async_copy(src_hbm.at[idx_v.at[b, k]], taps_v.at[b, k],
                             gsems[b])

    def drain_gathers(b):
        for k in range(4):
            pltpu.make_async_copy(src_hbm.at[idx_v.at[b, k]], taps_v.at[b, k],
                                  gsems[b]).wait()

    def combine(ci, b):
        base = worker_base + ci * CHUNK

        def px_body(g, c2):
            gsl = pl.ds(g * LANES, LANES)
            w00v = wgt_v[b, 0, gsl]
            w01v = wgt_v[b, 1, gsl]
            w10v = wgt_v[b, 2, gsl]
            w11v = wgt_v[b, 3, gsl]
            for jj in range(LANES):
                p = g * LANES + jj
                b00 = jnp.full((LANES,), w00v[jj], jnp.float32)
                b01 = jnp.full((LANES,), w01v[jj], jnp.float32)
                b10 = jnp.full((LANES,), w10v[jj], jnp.float32)
                b11 = jnp.full((LANES,), w11v[jj], jnp.float32)
                for cc in range(C // LANES):
                    sl = pl.ds(cc * LANES, LANES)
                    out_v[b, p, sl] = (
                        taps_v[b, 0, p, sl] * b00 + taps_v[b, 1, p, sl] * b01
                        + taps_v[b, 2, p, sl] * b10 + taps_v[b, 3, p, sl] * b11)
            return c2

        lax.fori_loop(0, CHUNK // LANES, px_body, 0, unroll=False)
        pltpu.async_copy(out_v.at[b], out_hbm.at[pl.ds(base, CHUNK)],
                         osems[b])

    def drain_store(b):
        pltpu.make_async_copy(out_v.at[b], out_hbm.at[pl.ds(0, CHUNK)],
                              osems[b]).wait()

    stage(0, 0)

    def pair_body(i, carry):
        for b in range(2):
            ci = 2 * i + b
            nb = 1 - b

            @pl.when(ci + 1 < NCHUNK)
            def _():
                stage(ci + 1, nb)

            drain_gathers(b)

            @pl.when(ci >= 2)
            def _():
                drain_store(b)

            combine(ci, b)
        return carry

    lax.fori_loop(0, NCHUNK // 2, pair_body, 0, unroll=False)
    drain_store(0)
    drain_store(1)


@jax.jit
def _warp(src_cl, fx, fy):
    mesh = plsc.VectorSubcoreMesh(core_axis_name="c", subcore_axis_name="s",
                                  num_cores=2, num_subcores=16)
    f = functools.partial(
        pl.kernel,
        out_type=jax.ShapeDtypeStruct((N, C), jnp.float32),
        mesh=mesh,
        scratch_types=[
            pltpu.VMEM((PIX_PER_W,), jnp.float32),
            pltpu.VMEM((PIX_PER_W,), jnp.float32),
            pltpu.VMEM((2, 4, CHUNK), jnp.int32),
            pltpu.VMEM((2, 4, CHUNK), jnp.float32),
            pltpu.VMEM((2, 4, CHUNK, CP), jnp.float32),
            pltpu.VMEM((2, CHUNK, C), jnp.float32),
            pltpu.SemaphoreType.DMA,
            pltpu.SemaphoreType.DMA,
            pltpu.SemaphoreType.DMA,
            pltpu.SemaphoreType.DMA,
        ],
    )(_warp_body)
    return f(src_cl, fx, fy)


def kernel(src, flow):
    src_p = jnp.pad(src, ((0, 0), (0, CP - C), (0, 0), (0, 0)))
    src_cl = src_p.transpose(0, 2, 3, 1).reshape(N, CP)
    fx = flow[..., 0].reshape(N)
    fy = flow[..., 1].reshape(N)
    out_cl = _warp(src_cl, fx, fy)
    return out_cl.reshape(B, H, W, C).transpose(0, 3, 1, 2)
